# baseline (speedup 1.0000x reference)
import functools

import jax
import jax.numpy as jnp
from jax import lax
from jax.experimental import pallas as pl
from jax.experimental.pallas import tpu as pltpu
from jax.experimental.pallas import tpu_sc as plsc

N = 31744
E = 507904
D = 128
NT = 16
NSC = 2
FC = 32
NCH = 4
ER = E // 128
BATCHN = 512
NPT = N // NT

@functools.lru_cache(maxsize=None)
def _build_deg():
  mesh = plsc.VectorSubcoreMesh(core_axis_name="c", subcore_axis_name="s")
  RW = ER // (NT * NSC)

  @functools.partial(
      pl.kernel,
      mesh=mesh,
      out_type=jax.ShapeDtypeStruct((NSC, N), jnp.float32),
      scratch_types=[
          pltpu.VMEM((RW, 128), jnp.int32),
          pltpu.VMEM((128,), jnp.float32),
          pltpu.VMEM_SHARED((N,), jnp.float32),
          pltpu.SemaphoreType.DMA,
      ],
      compiler_params=pltpu.CompilerParams(
          use_tc_tiling_on_sc=False, disable_bounds_checks=True),
  )
  def _deg_kernel(row_hbm, zeros_hbm, ones_hbm, deg_hbm,
                  row_v, ones_v, deg_sh, sem):
    c = lax.axis_index("c")
    s = lax.axis_index("s")
    w = s * NSC + c
    @pl.when(s == 0)
    def _():
      pltpu.sync_copy(zeros_hbm, deg_sh)
    pltpu.sync_copy(ones_hbm, ones_v)
    pltpu.sync_copy(row_hbm.at[pl.ds(w * RW, RW)], row_v)
    plsc.subcore_barrier()

    def body(i, _):
      for j in range(4):
        pltpu.async_copy(ones_v, deg_sh.at[row_v.at[i * 4 + j]], sem,
                         add=True)
      return 0

    lax.fori_loop(0, RW // 4, body, 0)

    def drain(i, _):
      pltpu.make_async_copy(ones_hbm, ones_v, sem).wait()
      return 0

    lax.fori_loop(0, RW, drain, 0)
    plsc.subcore_barrier()

    @pl.when(s == 0)
    def _():
      pltpu.sync_copy(deg_sh, deg_hbm.at[c])

  return _deg_kernel


@functools.lru_cache(maxsize=None)
def _build_prop():
  mesh = plsc.VectorSubcoreMesh(core_axis_name="c", subcore_axis_name="s")

  RPT = ER // NT
  BR = 4
  HB = RPT // 2
  NBLK = HB // BR

  @functools.partial(
      pl.kernel,
      mesh=mesh,
      out_type=jax.ShapeDtypeStruct((NCH, N, FC), jnp.float32),
      scratch_types=[
          pltpu.VMEM((HB, 128), jnp.int32),
          pltpu.VMEM((HB, 128), jnp.int32),
          pltpu.VMEM((BR, 128, FC), jnp.float32),
          pltpu.VMEM((BR, 128, FC), jnp.float32),
          pltpu.VMEM_SHARED((N, FC), jnp.float32),
          pltpu.SemaphoreType.DMA,
          pltpu.SemaphoreType.DMA,
          pltpu.SemaphoreType.DMA,
          pltpu.SemaphoreType.DMA,
      ],
      compiler_params=pltpu.CompilerParams(
          use_tc_tiling_on_sc=False, disable_bounds_checks=True),
  )
  def _prop_kernel(row_hbm, col_hbm, z_hbm, zeros_hbm, s_hbm,
                   row_v, col_v, gat_a, gat_b, acc_sh,
                   sem_a, sem_b, sem_sa, sem_sb):
    c = lax.axis_index("c")
    s = lax.axis_index("s")
    base = s * RPT

    for ch in range(2):
      pltpu.sync_copy(zeros_hbm, acc_sh.at[pl.ds(s * NPT, NPT)])
      plsc.subcore_barrier()
      zc = z_hbm.at[2 * c + ch]

      def fire(rb, buf, sem):
        return [pltpu.async_copy(zc.at[row_v.at[rb + j]], buf.at[j], sem)
                for j in range(BR)]

      def scat(rb, buf, sem):
        return [pltpu.async_copy(buf.at[j], acc_sh.at[col_v.at[rb + j]], sem,
                                 add=True)
                for j in range(BR)]

      for half in range(2):
        hbase = base + half * HB
        pltpu.sync_copy(row_hbm.at[pl.ds(hbase, HB)], row_v)
        pltpu.sync_copy(col_hbm.at[pl.ds(hbase, HB)], col_v)

        ha = fire(0, gat_a, sem_a)
        hb = fire(BR, gat_b, sem_b)

        def body(i, _):
          rb = 2 * i * BR
          for h in ha:
            h.wait()
          hsa = scat(rb, gat_a, sem_sa)
          for h in hb:
            h.wait()
          hsb = scat(rb + BR, gat_b, sem_sb)
          for h in hsa:
            h.wait()
          fire(rb + 2 * BR, gat_a, sem_a)
          for h in hsb:
            h.wait()
          fire(rb + 3 * BR, gat_b, sem_b)
          return 0

        lax.fori_loop(0, NBLK // 2 - 1, body, 0)
        tb = (NBLK - 3) * BR
        for h in ha:
          h.wait()
        hsa = scat(tb, gat_a, sem_sa)
        for h in hb:
          h.wait()
        hsb = scat(tb + BR, gat_b, sem_sb)
        for h in hsa:
          h.wait()
        fire(tb + 2 * BR, gat_a, sem_a)
        for h in hsb:
          h.wait()
        for h in ha:
          h.wait()
        for h in scat(tb + 2 * BR, gat_a, sem_sa):
          h.wait()

      plsc.subcore_barrier()
      pltpu.sync_copy(acc_sh.at[pl.ds(s * NPT, NPT)],
                      s_hbm.at[2 * c + ch].at[pl.ds(s * NPT, NPT)])
      plsc.subcore_barrier()

  return _prop_kernel


def _z_body(x_ref, dega_ref, degb_ref, w1_ref, z_ref):
    deg = dega_ref[...] + degb_ref[...]
    dis = jnp.where(deg > 0, lax.rsqrt(jnp.maximum(deg, 1e-12)), 0.0)
    xs = x_ref[...] * dis
    z = lax.dot_general(xs, w1_ref[...], (((1,), (1,)), ((), ())),
                        preferred_element_type=jnp.float32)
    q = z.shape[0] // 4
    for k in range(NCH):
        zc = z[:, k * FC:(k + 1) * FC]
        z_ref[k] = jnp.concatenate(
            [zc[a * q:(a + 1) * q, :] for a in range(4)], axis=1)


def _z_call(x, dega, degb, w1):
    nb = 16
    bs = N // nb
    return pl.pallas_call(
        _z_body,
        grid=(nb,),
        in_specs=[
            pl.BlockSpec((bs, D), lambda i: (i, 0)),
            pl.BlockSpec((bs, 1), lambda i: (i, 0)),
            pl.BlockSpec((bs, 1), lambda i: (i, 0)),
            pl.BlockSpec((D, D), lambda i: (0, 0)),
        ],
        out_specs=pl.BlockSpec((NCH, bs // 4, 4 * FC), lambda i: (0, i, 0)),
        out_shape=jax.ShapeDtypeStruct((NCH, N // 4, 4 * FC), jnp.float32),
    )(x, dega, degb, w1)


def _hc_body(x_ref, dega_ref, degb_ref, s_ref, w0_ref, b0_ref, wt_ref, bc_ref,
             o_ref):
    deg = dega_ref[...] + degb_ref[...]
    dis = jnp.where(deg > 0, lax.rsqrt(jnp.maximum(deg, 1e-12)), 0.0)
    chunks = []
    for k in range(NCH):
        p = s_ref[k]
        chunks.append(jnp.concatenate(
            [p[:, a * FC:(a + 1) * FC] for a in range(4)], axis=0))
    sfull = jnp.concatenate(chunks, axis=1)
    y = lax.dot_general(x_ref[...], w0_ref[...], (((1,), (1,)), ((), ())),
                        preferred_element_type=jnp.float32)
    h = jnp.maximum(y - dis * sfull + b0_ref[...], 0.0)
    cols = [jnp.sum(h * wt_ref[k], axis=1, keepdims=True) for k in range(3)]
    t = jnp.concatenate(cols, axis=1)
    rows = t.shape[0]
    nb = rows // 62
    r_idx = lax.broadcasted_iota(jnp.int32, (nb, rows), 1)
    b_idx = lax.broadcasted_iota(jnp.int32, (nb, rows), 0)
    g = jnp.where(r_idx // 62 == b_idx, 1.0, 0.0)
    logits = lax.dot_general(g, t, (((1,), (0,)), ((), ())),
                             preferred_element_type=jnp.float32)
    logits = logits + bc_ref[...]
    m = jnp.max(logits, axis=1, keepdims=True)
    e = jnp.exp(logits - m)
    o_ref[...] = e / jnp.sum(e, axis=1, keepdims=True)


def _hc_call(x, dega, degb, s, w0, b0, wt, bc):
    nb = 16
    bs = N // nb
    return pl.pallas_call(
        _hc_body,
        grid=(nb,),
        in_specs=[
            pl.BlockSpec((bs, D), lambda i: (i, 0)),
            pl.BlockSpec((bs, 1), lambda i: (i, 0)),
            pl.BlockSpec((bs, 1), lambda i: (i, 0)),
            pl.BlockSpec((NCH, bs // 4, 4 * FC), lambda i: (0, i, 0)),
            pl.BlockSpec((D, D), lambda i: (0, 0)),
            pl.BlockSpec((1, D), lambda i: (0, 0)),
            pl.BlockSpec((3, bs, D), lambda i: (0, 0, 0)),
            pl.BlockSpec((1, 3), lambda i: (0, 0)),
        ],
        out_specs=pl.BlockSpec((bs // 62, 3), lambda i: (i, 0)),
        out_shape=jax.ShapeDtypeStruct((BATCHN, 3), jnp.float32),
    )(x, dega, degb, s, w0, b0, wt, bc)


def kernel(x, edge_index, W0, W1, b0, Wc, bc):
    row0 = edge_index[0].astype(jnp.int32)
    col0 = edge_index[1].astype(jnp.int32)
    row = row0.reshape(ER, 128)
    bs = N // 16
    q = bs // 4

    def _perm(n):
        blk = n // bs
        loc = n % bs
        return blk * bs + 4 * (loc % q) + loc // q

    rowp = _perm(row0).reshape(ER, 128)
    colp = _perm(col0).reshape(ER, 128)

    zeros_n = jnp.zeros((N,), jnp.float32)
    ones128 = jnp.ones((128,), jnp.float32)
    deg2 = _build_deg()(row, zeros_n, ones128)
    dega = deg2[0].reshape(N, 1)
    degb = deg2[1].reshape(N, 1)

    zp = _z_call(x, dega, degb, W1)
    z = zp.reshape(NCH, N, FC)

    zeros_nf = jnp.zeros((NPT, FC), jnp.float32)
    s = _build_prop()(rowp, colp, z, zeros_nf)
    s4 = s.reshape(NCH, N // 4, 4 * FC)

    wt = jnp.tile(Wc.reshape(3, 62, D), (1, (N // 16) // 62, 1))
    return _hc_call(x, dega, degb, s4, W0, b0.reshape(1, D), wt,
                    bc.reshape(1, 3))


# speedup vs baseline: 1.1524x; 1.1524x over previous
import functools

import jax
import jax.numpy as jnp
from jax import lax
from jax.experimental import pallas as pl
from jax.experimental.pallas import tpu as pltpu
from jax.experimental.pallas import tpu_sc as plsc

N = 31744
E = 507904
D = 128
NT = 16
NSC = 2
FC = 32
NCH = 4
ER = E // 128
BATCHN = 512
NPT = N // NT

@functools.lru_cache(maxsize=None)
def _build_deg():
  mesh = plsc.VectorSubcoreMesh(core_axis_name="c", subcore_axis_name="s")
  RW = ER // (NT * NSC)

  @functools.partial(
      pl.kernel,
      mesh=mesh,
      out_type=jax.ShapeDtypeStruct((NSC, N), jnp.float32),
      scratch_types=[
          pltpu.VMEM((RW, 128), jnp.int32),
          pltpu.VMEM((128,), jnp.float32),
          pltpu.VMEM_SHARED((N,), jnp.float32),
          pltpu.SemaphoreType.DMA,
      ],
      compiler_params=pltpu.CompilerParams(
          use_tc_tiling_on_sc=False, disable_bounds_checks=True),
  )
  def _deg_kernel(row_hbm, zeros_hbm, ones_hbm, deg_hbm,
                  row_v, ones_v, deg_sh, sem):
    c = lax.axis_index("c")
    s = lax.axis_index("s")
    w = s * NSC + c
    @pl.when(s == 0)
    def _():
      pltpu.sync_copy(zeros_hbm, deg_sh)
    pltpu.sync_copy(ones_hbm, ones_v)
    pltpu.sync_copy(row_hbm.at[pl.ds(w * RW, RW)], row_v)
    plsc.subcore_barrier()

    def body(i, _):
      for j in range(4):
        pltpu.async_copy(ones_v, deg_sh.at[row_v.at[i * 4 + j]], sem,
                         add=True)
      return 0

    lax.fori_loop(0, RW // 4, body, 0)

    def drain(i, _):
      pltpu.make_async_copy(ones_hbm, ones_v, sem).wait()
      return 0

    lax.fori_loop(0, RW, drain, 0)
    plsc.subcore_barrier()

    @pl.when(s == 0)
    def _():
      pltpu.sync_copy(deg_sh, deg_hbm.at[c])

  return _deg_kernel


@functools.lru_cache(maxsize=None)
def _build_prop():
  mesh = plsc.VectorSubcoreMesh(core_axis_name="c", subcore_axis_name="s")

  RPT = ER // NT
  BR = 4
  HB = RPT // 2
  NBLK = HB // BR

  @functools.partial(
      pl.kernel,
      mesh=mesh,
      out_type=jax.ShapeDtypeStruct((NCH, N, FC), jnp.float32),
      scratch_types=[
          pltpu.VMEM((HB, 128), jnp.int32),
          pltpu.VMEM((HB, 128), jnp.int32),
          pltpu.VMEM((BR, 128, FC), jnp.float32),
          pltpu.VMEM((BR, 128, FC), jnp.float32),
          pltpu.VMEM_SHARED((N, FC), jnp.float32),
          pltpu.SemaphoreType.DMA,
          pltpu.SemaphoreType.DMA,
      ],
      compiler_params=pltpu.CompilerParams(
          use_tc_tiling_on_sc=False, disable_bounds_checks=True),
  )
  def _prop_kernel(row_hbm, col_hbm, z_hbm, zeros_hbm, s_hbm,
                   row_v, col_v, gat_a, gat_b, acc_sh, sem_a, sem_b):
    c = lax.axis_index("c")
    s = lax.axis_index("s")
    base = s * RPT

    for ch in range(2):
      pltpu.sync_copy(zeros_hbm, acc_sh.at[pl.ds(s * NPT, NPT)])
      plsc.subcore_barrier()
      zc = z_hbm.at[2 * c + ch]

      def fire(rb, buf, sem):
        return [pltpu.async_copy(zc.at[row_v.at[rb + j]], buf.at[j], sem)
                for j in range(BR)]

      def scat(rb, buf):
        for j in range(BR):
          pltpu.sync_copy(buf.at[j], acc_sh.at[col_v.at[rb + j]], add=True)

      for half in range(2):
        hbase = base + half * HB
        pltpu.sync_copy(row_hbm.at[pl.ds(hbase, HB)], row_v)
        pltpu.sync_copy(col_hbm.at[pl.ds(hbase, HB)], col_v)

        ha = fire(0, gat_a, sem_a)
        hb = fire(BR, gat_b, sem_b)

        def body(i, _):
          rb = 2 * i * BR
          for h in ha:
            h.wait()
          scat(rb, gat_a)
          fire(rb + 2 * BR, gat_a, sem_a)
          for h in hb:
            h.wait()
          scat(rb + BR, gat_b)
          fire(rb + 3 * BR, gat_b, sem_b)
          return 0

        lax.fori_loop(0, NBLK // 2 - 1, body, 0)
        tb = (NBLK - 3) * BR
        for h in ha:
          h.wait()
        scat(tb, gat_a)
        fire(tb + 2 * BR, gat_a, sem_a)
        for h in hb:
          h.wait()
        scat(tb + BR, gat_b)
        for h in ha:
          h.wait()
        scat(tb + 2 * BR, gat_a)

      plsc.subcore_barrier()
      pltpu.sync_copy(acc_sh.at[pl.ds(s * NPT, NPT)],
                      s_hbm.at[2 * c + ch].at[pl.ds(s * NPT, NPT)])
      plsc.subcore_barrier()

  return _prop_kernel


def _z_body(x_ref, dega_ref, degb_ref, w1_ref, z_ref):
    deg = dega_ref[...] + degb_ref[...]
    dis = jnp.where(deg > 0, lax.rsqrt(jnp.maximum(deg, 1e-12)), 0.0)
    xs = x_ref[...] * dis
    z = lax.dot_general(xs, w1_ref[...], (((1,), (1,)), ((), ())),
                        preferred_element_type=jnp.float32)
    q = z.shape[0] // 4
    for k in range(NCH):
        zc = z[:, k * FC:(k + 1) * FC]
        z_ref[k] = jnp.concatenate(
            [zc[a * q:(a + 1) * q, :] for a in range(4)], axis=1)


def _z_call(x, dega, degb, w1):
    nb = 16
    bs = N // nb
    return pl.pallas_call(
        _z_body,
        grid=(nb,),
        in_specs=[
            pl.BlockSpec((bs, D), lambda i: (i, 0)),
            pl.BlockSpec((bs, 1), lambda i: (i, 0)),
            pl.BlockSpec((bs, 1), lambda i: (i, 0)),
            pl.BlockSpec((D, D), lambda i: (0, 0)),
        ],
        out_specs=pl.BlockSpec((NCH, bs // 4, 4 * FC), lambda i: (0, i, 0)),
        out_shape=jax.ShapeDtypeStruct((NCH, N // 4, 4 * FC), jnp.float32),
    )(x, dega, degb, w1)


def _hc_body(x_ref, dega_ref, degb_ref, s_ref, w0_ref, b0_ref, wt_ref, bc_ref,
             o_ref):
    deg = dega_ref[...] + degb_ref[...]
    dis = jnp.where(deg > 0, lax.rsqrt(jnp.maximum(deg, 1e-12)), 0.0)
    chunks = []
    for k in range(NCH):
        p = s_ref[k]
        chunks.append(jnp.concatenate(
            [p[:, a * FC:(a + 1) * FC] for a in range(4)], axis=0))
    sfull = jnp.concatenate(chunks, axis=1)
    y = lax.dot_general(x_ref[...], w0_ref[...], (((1,), (1,)), ((), ())),
                        preferred_element_type=jnp.float32)
    h = jnp.maximum(y - dis * sfull + b0_ref[...], 0.0)
    cols = [jnp.sum(h * wt_ref[k], axis=1, keepdims=True) for k in range(3)]
    t = jnp.concatenate(cols, axis=1)
    rows = t.shape[0]
    nb = rows // 62
    r_idx = lax.broadcasted_iota(jnp.int32, (nb, rows), 1)
    b_idx = lax.broadcasted_iota(jnp.int32, (nb, rows), 0)
    g = jnp.where(r_idx // 62 == b_idx, 1.0, 0.0)
    logits = lax.dot_general(g, t, (((1,), (0,)), ((), ())),
                             preferred_element_type=jnp.float32)
    logits = logits + bc_ref[...]
    m = jnp.max(logits, axis=1, keepdims=True)
    e = jnp.exp(logits - m)
    o_ref[...] = e / jnp.sum(e, axis=1, keepdims=True)


def _hc_call(x, dega, degb, s, w0, b0, wt, bc):
    nb = 16
    bs = N // nb
    return pl.pallas_call(
        _hc_body,
        grid=(nb,),
        in_specs=[
            pl.BlockSpec((bs, D), lambda i: (i, 0)),
            pl.BlockSpec((bs, 1), lambda i: (i, 0)),
            pl.BlockSpec((bs, 1), lambda i: (i, 0)),
            pl.BlockSpec((NCH, bs // 4, 4 * FC), lambda i: (0, i, 0)),
            pl.BlockSpec((D, D), lambda i: (0, 0)),
            pl.BlockSpec((1, D), lambda i: (0, 0)),
            pl.BlockSpec((3, bs, D), lambda i: (0, 0, 0)),
            pl.BlockSpec((1, 3), lambda i: (0, 0)),
        ],
        out_specs=pl.BlockSpec((bs // 62, 3), lambda i: (i, 0)),
        out_shape=jax.ShapeDtypeStruct((BATCHN, 3), jnp.float32),
    )(x, dega, degb, s, w0, b0, wt, bc)


def kernel(x, edge_index, W0, W1, b0, Wc, bc):
    row0 = edge_index[0].astype(jnp.int32)
    col0 = edge_index[1].astype(jnp.int32)
    row = row0.reshape(ER, 128)
    bs = N // 16
    q = bs // 4

    def _perm(n):
        blk = n // bs
        loc = n % bs
        return blk * bs + 4 * (loc % q) + loc // q

    rowp = _perm(row0).reshape(ER, 128)
    colp = _perm(col0).reshape(ER, 128)

    zeros_n = jnp.zeros((N,), jnp.float32)
    ones128 = jnp.ones((128,), jnp.float32)
    deg2 = _build_deg()(row, zeros_n, ones128)
    dega = deg2[0].reshape(N, 1)
    degb = deg2[1].reshape(N, 1)

    zp = _z_call(x, dega, degb, W1)
    z = zp.reshape(NCH, N, FC)

    zeros_nf = jnp.zeros((NPT, FC), jnp.float32)
    s = _build_prop()(rowp, colp, z, zeros_nf)
    s4 = s.reshape(NCH, N // 4, 4 * FC)

    wt = jnp.tile(Wc.reshape(3, 62, D), (1, (N // 16) // 62, 1))
    return _hc_call(x, dega, degb, s4, W0, b0.reshape(1, D), wt,
                    bc.reshape(1, 3))


# speedup vs baseline: 1.2815x; 1.1120x over previous
import functools

import jax
import jax.numpy as jnp
from jax import lax
from jax.experimental import pallas as pl
from jax.experimental.pallas import tpu as pltpu
from jax.experimental.pallas import tpu_sc as plsc

N = 31744
E = 507904
D = 128
NT = 16
NSC = 2
FC = 64
NCH = 2
ER = E // 128
BATCHN = 512
NPT = N // NT

@functools.lru_cache(maxsize=None)
def _build_deg():
  mesh = plsc.VectorSubcoreMesh(core_axis_name="c", subcore_axis_name="s")
  RW = ER // (NT * NSC)

  @functools.partial(
      pl.kernel,
      mesh=mesh,
      out_type=jax.ShapeDtypeStruct((NSC, N), jnp.float32),
      scratch_types=[
          pltpu.VMEM((RW, 128), jnp.int32),
          pltpu.VMEM((128,), jnp.float32),
          pltpu.VMEM_SHARED((N,), jnp.float32),
          pltpu.SemaphoreType.DMA,
      ],
      compiler_params=pltpu.CompilerParams(
          use_tc_tiling_on_sc=False, disable_bounds_checks=True),
  )
  def _deg_kernel(row_hbm, zeros_hbm, ones_hbm, deg_hbm,
                  row_v, ones_v, deg_sh, sem):
    c = lax.axis_index("c")
    s = lax.axis_index("s")
    w = s * NSC + c
    @pl.when(s == 0)
    def _():
      pltpu.sync_copy(zeros_hbm, deg_sh)
    pltpu.sync_copy(ones_hbm, ones_v)
    pltpu.sync_copy(row_hbm.at[pl.ds(w * RW, RW)], row_v)
    plsc.subcore_barrier()

    def body(i, _):
      for j in range(4):
        pltpu.async_copy(ones_v, deg_sh.at[row_v.at[i * 4 + j]], sem,
                         add=True)
      return 0

    lax.fori_loop(0, RW // 4, body, 0)

    def drain(i, _):
      pltpu.make_async_copy(ones_hbm, ones_v, sem).wait()
      return 0

    lax.fori_loop(0, RW, drain, 0)
    plsc.subcore_barrier()

    @pl.when(s == 0)
    def _():
      pltpu.sync_copy(deg_sh, deg_hbm.at[c])

  return _deg_kernel


@functools.lru_cache(maxsize=None)
def _build_prop():
  mesh = plsc.VectorSubcoreMesh(core_axis_name="c", subcore_axis_name="s")

  RPT = ER // NT
  BR = 4
  HB = RPT // 2
  NBLK = HB // BR

  @functools.partial(
      pl.kernel,
      mesh=mesh,
      out_type=jax.ShapeDtypeStruct((NSC, N, FC), jnp.bfloat16),
      scratch_types=[
          pltpu.VMEM((HB, 128), jnp.int32),
          pltpu.VMEM((HB, 128), jnp.int32),
          pltpu.VMEM((BR, 128, FC), jnp.bfloat16),
          pltpu.VMEM((BR, 128, FC), jnp.bfloat16),
          pltpu.VMEM_SHARED((N, FC), jnp.bfloat16),
          pltpu.SemaphoreType.DMA,
          pltpu.SemaphoreType.DMA,
      ],
      compiler_params=pltpu.CompilerParams(
          use_tc_tiling_on_sc=False, disable_bounds_checks=True),
  )
  def _prop_kernel(row_hbm, col_hbm, z_hbm, zeros_hbm, s_hbm,
                   row_v, col_v, gat_a, gat_b, acc_sh, sem_a, sem_b):
    c = lax.axis_index("c")
    s = lax.axis_index("s")
    base = s * RPT

    pltpu.sync_copy(zeros_hbm, acc_sh.at[pl.ds(s * NPT, NPT)])
    plsc.subcore_barrier()
    zc = z_hbm.at[c]

    def fire(rb, buf, sem):
      return [pltpu.async_copy(zc.at[row_v.at[rb + j]], buf.at[j], sem)
              for j in range(BR)]

    def scat(rb, buf):
      for j in range(BR):
        pltpu.sync_copy(buf.at[j], acc_sh.at[col_v.at[rb + j]], add=True)

    for half in range(2):
      hbase = base + half * HB
      pltpu.sync_copy(row_hbm.at[pl.ds(hbase, HB)], row_v)
      pltpu.sync_copy(col_hbm.at[pl.ds(hbase, HB)], col_v)

      ha = fire(0, gat_a, sem_a)
      hb = fire(BR, gat_b, sem_b)

      def body(i, _):
        rb = 2 * i * BR
        for h in ha:
          h.wait()
        scat(rb, gat_a)
        fire(rb + 2 * BR, gat_a, sem_a)
        for h in hb:
          h.wait()
        scat(rb + BR, gat_b)
        fire(rb + 3 * BR, gat_b, sem_b)
        return 0

      lax.fori_loop(0, NBLK // 2 - 1, body, 0)
      tb = (NBLK - 3) * BR
      for h in ha:
        h.wait()
      scat(tb, gat_a)
      fire(tb + 2 * BR, gat_a, sem_a)
      for h in hb:
        h.wait()
      scat(tb + BR, gat_b)
      for h in ha:
        h.wait()
      scat(tb + 2 * BR, gat_a)

    plsc.subcore_barrier()
    pltpu.sync_copy(acc_sh.at[pl.ds(s * NPT, NPT)],
                    s_hbm.at[c].at[pl.ds(s * NPT, NPT)])

  return _prop_kernel


def _z_body(x_ref, dega_ref, degb_ref, w1_ref, z_ref):
    deg = dega_ref[...] + degb_ref[...]
    dis = jnp.where(deg > 0, lax.rsqrt(jnp.maximum(deg, 1e-12)), 0.0)
    xs = x_ref[...] * dis
    z = lax.dot_general(xs, w1_ref[...], (((1,), (1,)), ((), ())),
                        preferred_element_type=jnp.float32)
    zb = z.astype(jnp.bfloat16)
    q = z.shape[0] // 2
    for k in range(NCH):
        zc = zb[:, k * FC:(k + 1) * FC]
        z_ref[k] = jnp.concatenate(
            [zc[a * q:(a + 1) * q, :] for a in range(2)], axis=1)


def _z_call(x, dega, degb, w1):
    nb = 16
    bs = N // nb
    return pl.pallas_call(
        _z_body,
        grid=(nb,),
        in_specs=[
            pl.BlockSpec((bs, D), lambda i: (i, 0)),
            pl.BlockSpec((bs, 1), lambda i: (i, 0)),
            pl.BlockSpec((bs, 1), lambda i: (i, 0)),
            pl.BlockSpec((D, D), lambda i: (0, 0)),
        ],
        out_specs=pl.BlockSpec((NCH, bs // 2, 2 * FC), lambda i: (0, i, 0)),
        out_shape=jax.ShapeDtypeStruct((NCH, N // 2, 2 * FC), jnp.bfloat16),
    )(x, dega, degb, w1)


def _hc_body(x_ref, dega_ref, degb_ref, s_ref, w0_ref, b0_ref, wt_ref, bc_ref,
             o_ref):
    deg = dega_ref[...] + degb_ref[...]
    dis = jnp.where(deg > 0, lax.rsqrt(jnp.maximum(deg, 1e-12)), 0.0)
    chunks = []
    for k in range(NCH):
        p = s_ref[k]
        chunks.append(jnp.concatenate(
            [p[:, a * FC:(a + 1) * FC] for a in range(2)], axis=0))
    sfull = jnp.concatenate(chunks, axis=1).astype(jnp.float32)
    y = lax.dot_general(x_ref[...], w0_ref[...], (((1,), (1,)), ((), ())),
                        preferred_element_type=jnp.float32)
    h = jnp.maximum(y - dis * sfull + b0_ref[...], 0.0)
    cols = [jnp.sum(h * wt_ref[k], axis=1, keepdims=True) for k in range(3)]
    t = jnp.concatenate(cols, axis=1)
    rows = t.shape[0]
    nb = rows // 62
    r_idx = lax.broadcasted_iota(jnp.int32, (nb, rows), 1)
    b_idx = lax.broadcasted_iota(jnp.int32, (nb, rows), 0)
    g = jnp.where(r_idx // 62 == b_idx, 1.0, 0.0)
    logits = lax.dot_general(g, t, (((1,), (0,)), ((), ())),
                             preferred_element_type=jnp.float32)
    logits = logits + bc_ref[...]
    m = jnp.max(logits, axis=1, keepdims=True)
    e = jnp.exp(logits - m)
    o_ref[...] = e / jnp.sum(e, axis=1, keepdims=True)


def _hc_call(x, dega, degb, s, w0, b0, wt, bc):
    nb = 16
    bs = N // nb
    return pl.pallas_call(
        _hc_body,
        grid=(nb,),
        in_specs=[
            pl.BlockSpec((bs, D), lambda i: (i, 0)),
            pl.BlockSpec((bs, 1), lambda i: (i, 0)),
            pl.BlockSpec((bs, 1), lambda i: (i, 0)),
            pl.BlockSpec((NCH, bs // 2, 2 * FC), lambda i: (0, i, 0)),
            pl.BlockSpec((D, D), lambda i: (0, 0)),
            pl.BlockSpec((1, D), lambda i: (0, 0)),
            pl.BlockSpec((3, bs, D), lambda i: (0, 0, 0)),
            pl.BlockSpec((1, 3), lambda i: (0, 0)),
        ],
        out_specs=pl.BlockSpec((bs // 62, 3), lambda i: (i, 0)),
        out_shape=jax.ShapeDtypeStruct((BATCHN, 3), jnp.float32),
    )(x, dega, degb, s, w0, b0, wt, bc)


def kernel(x, edge_index, W0, W1, b0, Wc, bc):
    row0 = edge_index[0].astype(jnp.int32)
    col0 = edge_index[1].astype(jnp.int32)
    row = row0.reshape(ER, 128)
    bs = N // 16
    q = bs // 2

    def _perm(n):
        blk = n // bs
        loc = n % bs
        return blk * bs + 2 * (loc % q) + loc // q

    rowp = _perm(row0).reshape(ER, 128)
    colp = _perm(col0).reshape(ER, 128)

    zeros_n = jnp.zeros((N,), jnp.float32)
    ones128 = jnp.ones((128,), jnp.float32)
    deg2 = _build_deg()(row, zeros_n, ones128)
    dega = deg2[0].reshape(N, 1)
    degb = deg2[1].reshape(N, 1)

    zp = _z_call(x, dega, degb, W1)
    z = zp.reshape(NCH, N, FC)

    zeros_nf = jnp.zeros((NPT, FC), jnp.bfloat16)
    s = _build_prop()(rowp, colp, z, zeros_nf)
    s4 = s.reshape(NCH, N // 2, 2 * FC)

    wt = jnp.tile(Wc.reshape(3, 62, D), (1, (N // 16) // 62, 1))
    return _hc_call(x, dega, degb, s4, W0, b0.reshape(1, D), wt,
                    bc.reshape(1, 3))


# speedup vs baseline: 1.3179x; 1.0284x over previous
import functools

import jax
import jax.numpy as jnp
from jax import lax
from jax.experimental import pallas as pl
from jax.experimental.pallas import tpu as pltpu
from jax.experimental.pallas import tpu_sc as plsc

N = 31744
E = 507904
D = 128
NT = 16
NSC = 2
FC = 64
NCH = 2
ER = E // 128
BATCHN = 512
NPT = N // NT

@functools.lru_cache(maxsize=None)
def _build_deg():
  mesh = plsc.VectorSubcoreMesh(core_axis_name="c", subcore_axis_name="s")
  RW = ER // (NT * NSC)

  @functools.partial(
      pl.kernel,
      mesh=mesh,
      out_type=jax.ShapeDtypeStruct((NSC, N), jnp.float32),
      scratch_types=[
          pltpu.VMEM((RW, 128), jnp.int32),
          pltpu.VMEM((128,), jnp.float32),
          pltpu.VMEM_SHARED((N,), jnp.float32),
          pltpu.SemaphoreType.DMA,
      ],
      compiler_params=pltpu.CompilerParams(
          use_tc_tiling_on_sc=False, disable_bounds_checks=True),
  )
  def _deg_kernel(row_hbm, zeros_hbm, ones_hbm, deg_hbm,
                  row_v, ones_v, deg_sh, sem):
    c = lax.axis_index("c")
    s = lax.axis_index("s")
    w = s * NSC + c
    @pl.when(s == 0)
    def _():
      pltpu.sync_copy(zeros_hbm, deg_sh)
    pltpu.sync_copy(ones_hbm, ones_v)
    pltpu.sync_copy(row_hbm.at[pl.ds(w * RW, RW)], row_v)
    plsc.subcore_barrier()

    def body(i, _):
      for j in range(4):
        pltpu.async_copy(ones_v, deg_sh.at[row_v.at[i * 4 + j]], sem,
                         add=True)
      return 0

    lax.fori_loop(0, RW // 4, body, 0)

    def drain(i, _):
      pltpu.make_async_copy(ones_hbm, ones_v, sem).wait()
      return 0

    lax.fori_loop(0, RW, drain, 0)
    plsc.subcore_barrier()

    @pl.when(s == 0)
    def _():
      pltpu.sync_copy(deg_sh, deg_hbm.at[c])

  return _deg_kernel


@functools.lru_cache(maxsize=None)
def _build_prop():
  mesh = plsc.VectorSubcoreMesh(core_axis_name="c", subcore_axis_name="s")

  RPT = ER // NT
  BR = 4
  HB = RPT // 2
  NBLK = HB // BR

  @functools.partial(
      pl.kernel,
      mesh=mesh,
      out_type=jax.ShapeDtypeStruct((NSC, N, FC), jnp.bfloat16),
      scratch_types=[
          pltpu.VMEM((HB, 128), jnp.int32),
          pltpu.VMEM((HB, 128), jnp.int32),
          pltpu.VMEM((BR, 128, FC), jnp.bfloat16),
          pltpu.VMEM((BR, 128, FC), jnp.bfloat16),
          pltpu.VMEM_SHARED((N, FC), jnp.bfloat16),
          pltpu.SemaphoreType.DMA,
          pltpu.SemaphoreType.DMA,
      ],
      compiler_params=pltpu.CompilerParams(
          use_tc_tiling_on_sc=False, disable_bounds_checks=True),
  )
  def _prop_kernel(row_hbm, col_hbm, z_hbm, zeros_hbm, s_hbm,
                   row_v, col_v, gat_a, gat_b, acc_sh, sem_a, sem_b):
    c = lax.axis_index("c")
    s = lax.axis_index("s")
    base = s * RPT

    pltpu.sync_copy(zeros_hbm, acc_sh.at[pl.ds(s * NPT, NPT)])
    plsc.subcore_barrier()
    zc = z_hbm.at[c]

    def fire(rb, buf, sem):
      return [pltpu.async_copy(zc.at[row_v.at[rb + j]], buf.at[j], sem)
              for j in range(BR)]

    def scat(rb, buf):
      for j in range(BR):
        pltpu.sync_copy(buf.at[j], acc_sh.at[col_v.at[rb + j]], add=True)

    for half in range(2):
      hbase = base + half * HB
      pltpu.sync_copy(row_hbm.at[pl.ds(hbase, HB)], row_v)
      pltpu.sync_copy(col_hbm.at[pl.ds(hbase, HB)], col_v)

      ha = fire(0, gat_a, sem_a)
      hb = fire(BR, gat_b, sem_b)

      def body(i, _):
        rb = 2 * i * BR
        for h in ha:
          h.wait()
        scat(rb, gat_a)
        fire(rb + 2 * BR, gat_a, sem_a)
        for h in hb:
          h.wait()
        scat(rb + BR, gat_b)
        fire(rb + 3 * BR, gat_b, sem_b)
        return 0

      lax.fori_loop(0, NBLK // 2 - 1, body, 0)
      tb = (NBLK - 3) * BR
      for h in ha:
        h.wait()
      scat(tb, gat_a)
      fire(tb + 2 * BR, gat_a, sem_a)
      for h in hb:
        h.wait()
      scat(tb + BR, gat_b)
      for h in ha:
        h.wait()
      scat(tb + 2 * BR, gat_a)

    plsc.subcore_barrier()
    pltpu.sync_copy(acc_sh.at[pl.ds(s * NPT, NPT)],
                    s_hbm.at[c].at[pl.ds(s * NPT, NPT)])

  return _prop_kernel


def _z_body(x_ref, dega_ref, degb_ref, w1_ref, z_ref):
    deg = dega_ref[...] + degb_ref[...]
    dis = jnp.where(deg > 0, lax.rsqrt(jnp.maximum(deg, 1e-12)), 0.0)
    xs = x_ref[...] * dis
    z = lax.dot_general(xs, w1_ref[...], (((1,), (1,)), ((), ())),
                        preferred_element_type=jnp.float32)
    zb = z.astype(jnp.bfloat16)
    for k in range(NCH):
        z_ref[k] = zb[:, k * FC:(k + 1) * FC]


def _z_call(x, dega, degb, w1):
    nb = 16
    bs = N // nb
    return pl.pallas_call(
        _z_body,
        grid=(nb,),
        in_specs=[
            pl.BlockSpec((bs, D), lambda i: (i, 0)),
            pl.BlockSpec((bs, 1), lambda i: (i, 0)),
            pl.BlockSpec((bs, 1), lambda i: (i, 0)),
            pl.BlockSpec((D, D), lambda i: (0, 0)),
        ],
        out_specs=pl.BlockSpec((NCH, bs, FC), lambda i: (0, i, 0)),
        out_shape=jax.ShapeDtypeStruct((NCH, N, FC), jnp.bfloat16),
    )(x, dega, degb, w1)


def _hc_body(x_ref, dega_ref, degb_ref, s_ref, w0_ref, b0_ref, wt_ref, bc_ref,
             o_ref):
    deg = dega_ref[...] + degb_ref[...]
    dis = jnp.where(deg > 0, lax.rsqrt(jnp.maximum(deg, 1e-12)), 0.0)
    sfull = jnp.concatenate(
        [s_ref[k] for k in range(NCH)], axis=1).astype(jnp.float32)
    y = lax.dot_general(x_ref[...], w0_ref[...], (((1,), (1,)), ((), ())),
                        preferred_element_type=jnp.float32)
    h = jnp.maximum(y - dis * sfull + b0_ref[...], 0.0)
    cols = [jnp.sum(h * wt_ref[k], axis=1, keepdims=True) for k in range(3)]
    t = jnp.concatenate(cols, axis=1)
    rows = t.shape[0]
    nb = rows // 62
    r_idx = lax.broadcasted_iota(jnp.int32, (nb, rows), 1)
    b_idx = lax.broadcasted_iota(jnp.int32, (nb, rows), 0)
    g = jnp.where(r_idx // 62 == b_idx, 1.0, 0.0)
    logits = lax.dot_general(g, t, (((1,), (0,)), ((), ())),
                             preferred_element_type=jnp.float32)
    logits = logits + bc_ref[...]
    m = jnp.max(logits, axis=1, keepdims=True)
    e = jnp.exp(logits - m)
    o_ref[...] = e / jnp.sum(e, axis=1, keepdims=True)


def _hc_call(x, dega, degb, s, w0, b0, wt, bc):
    nb = 16
    bs = N // nb
    return pl.pallas_call(
        _hc_body,
        grid=(nb,),
        in_specs=[
            pl.BlockSpec((bs, D), lambda i: (i, 0)),
            pl.BlockSpec((bs, 1), lambda i: (i, 0)),
            pl.BlockSpec((bs, 1), lambda i: (i, 0)),
            pl.BlockSpec((NCH, bs, FC), lambda i: (0, i, 0)),
            pl.BlockSpec((D, D), lambda i: (0, 0)),
            pl.BlockSpec((1, D), lambda i: (0, 0)),
            pl.BlockSpec((3, bs, D), lambda i: (0, 0, 0)),
            pl.BlockSpec((1, 3), lambda i: (0, 0)),
        ],
        out_specs=pl.BlockSpec((bs // 62, 3), lambda i: (i, 0)),
        out_shape=jax.ShapeDtypeStruct((BATCHN, 3), jnp.float32),
    )(x, dega, degb, s, w0, b0, wt, bc)


def kernel(x, edge_index, W0, W1, b0, Wc, bc):
    row = edge_index[0].astype(jnp.int32).reshape(ER, 128)
    col = edge_index[1].astype(jnp.int32).reshape(ER, 128)

    zeros_n = jnp.zeros((N,), jnp.float32)
    ones128 = jnp.ones((128,), jnp.float32)
    deg2 = _build_deg()(row, zeros_n, ones128)
    dega = deg2[0].reshape(N, 1)
    degb = deg2[1].reshape(N, 1)

    z = _z_call(x, dega, degb, W1)

    zeros_nf = jnp.zeros((NPT, FC), jnp.bfloat16)
    s = _build_prop()(row, col, z, zeros_nf)

    wt = jnp.tile(Wc.reshape(3, 62, D), (1, (N // 16) // 62, 1))
    return _hc_call(x, dega, degb, s, W0, b0.reshape(1, D), wt,
                    bc.reshape(1, 3))
